# Initial kernel scaffold; baseline (speedup 1.0000x reference)
import jax
import jax.numpy as jnp
from jax.experimental import pallas as pl


def kernel(input):
    raise NotImplementedError("write your pallas kernel here")


# speedup vs baseline: 1.5885x; 1.5885x over previous
import jax
import jax.numpy as jnp
from jax.experimental import pallas as pl

_ANCHORS = [(116.0, 90.0), (156.0, 198.0), (373.0, 326.0)]
_NUM_ANCHORS = 3
_NUM_CLASSES = 80
_BBOX_ATTRS = 5 + _NUM_CLASSES
_INPUT_SHAPE = (608, 608)


def _decode_kernel(in_ref, out_ref, *, in_h, in_w, stride_w, stride_h):
    hw = in_h * in_w
    a = pl.program_id(1)
    aw8 = jnp.where(a == 0, _ANCHORS[0][0] / stride_w,
                    jnp.where(a == 1, _ANCHORS[1][0] / stride_w,
                              _ANCHORS[2][0] / stride_w))
    ah8 = jnp.where(a == 0, _ANCHORS[0][1] / stride_h,
                    jnp.where(a == 1, _ANCHORS[1][1] / stride_h,
                              _ANCHORS[2][1] / stride_h))

    p = in_ref[0, 0]
    sig = jax.nn.sigmoid(p)
    ex = jnp.exp(p)

    r = jax.lax.broadcasted_iota(jnp.int32, (_BBOX_ATTRS, hw), 0)
    k = jax.lax.broadcasted_iota(jnp.int32, (_BBOX_ATTRS, hw), 1)
    gx = (k % in_w).astype(jnp.float32)
    gy = (k // in_w).astype(jnp.float32)

    val = jnp.where(
        r == 0, (sig + gx) * stride_w,
        jnp.where(
            r == 1, (sig + gy) * stride_h,
            jnp.where(
                r == 2, ex * (aw8 * stride_w),
                jnp.where(r == 3, ex * (ah8 * stride_h), sig))))
    out_ref[0, 0] = val.T


def kernel(input):
    bs, ch, in_h, in_w = input.shape
    hw = in_h * in_w
    stride_h = _INPUT_SHAPE[0] / in_h
    stride_w = _INPUT_SHAPE[1] / in_w

    x = input.reshape(bs, _NUM_ANCHORS, _BBOX_ATTRS, hw)

    out = pl.pallas_call(
        lambda i_ref, o_ref: _decode_kernel(
            i_ref, o_ref, in_h=in_h, in_w=in_w,
            stride_w=stride_w, stride_h=stride_h),
        grid=(bs, _NUM_ANCHORS),
        in_specs=[pl.BlockSpec((1, 1, _BBOX_ATTRS, hw), lambda b, a: (b, a, 0, 0))],
        out_specs=pl.BlockSpec((1, 1, hw, _BBOX_ATTRS), lambda b, a: (b, a, 0, 0)),
        out_shape=jax.ShapeDtypeStruct((bs, _NUM_ANCHORS, hw, _BBOX_ATTRS), jnp.float32),
    )(x)

    return out.reshape(bs, _NUM_ANCHORS * hw, _BBOX_ATTRS)
